# Initial kernel scaffold; baseline (speedup 1.0000x reference)
import functools

import jax
import jax.numpy as jnp
from jax import lax
from jax.experimental import pallas as pl
from jax.experimental.pallas import tpu as pltpu
from jax.experimental.pallas import tpu_sc as plsc


def _make_gather(b_total: int, d: int):
    info = plsc.get_sparse_core_info()
    nw = info.num_cores * info.num_subcores
    b_per_w = b_total // nw
    chunk = 1024
    n_chunks = b_per_w // chunk

    mesh = plsc.VectorSubcoreMesh(core_axis_name="c", subcore_axis_name="s")

    @functools.partial(
        pl.kernel,
        mesh=mesh,
        out_type=jax.ShapeDtypeStruct((b_total, d), jnp.float32),
        scratch_types=[
            pltpu.VMEM((b_per_w,), jnp.int32),
            pltpu.VMEM((2, chunk, d), jnp.float32),
            pltpu.SemaphoreType.DMA,
            pltpu.SemaphoreType.DMA,
        ],
    )
    def gather_kernel(table_hbm, idx_hbm, out_hbm, idx_v, rows_v, gsem, osem):
        wid = lax.axis_index("s") * info.num_cores + lax.axis_index("c")
        base = wid * b_per_w
        pltpu.sync_copy(idx_hbm.at[pl.ds(base, b_per_w)], idx_v)
        gathers = [None, None]
        outs = [None, None]
        gathers[0] = pltpu.async_copy(
            table_hbm.at[idx_v.at[pl.ds(0, chunk)]], rows_v.at[0], gsem
        )
        for c in range(n_chunks):
            cur = c % 2
            nxt = (c + 1) % 2
            if c + 1 < n_chunks:
                gathers[nxt] = pltpu.async_copy(
                    table_hbm.at[idx_v.at[pl.ds((c + 1) * chunk, chunk)]],
                    rows_v.at[nxt],
                    gsem,
                )
            gathers[cur].wait()
            if outs[cur] is not None:
                outs[cur].wait()
            outs[cur] = pltpu.async_copy(
                rows_v.at[cur],
                out_hbm.at[pl.ds(base + c * chunk, chunk)],
                osem,
            )
        for o in outs:
            if o is not None:
                o.wait()

    return gather_kernel


def kernel(inputs, indices):
    d = inputs.shape[1]
    idx_flat = indices.reshape(-1)
    out = _make_gather(idx_flat.shape[0], d)(inputs, idx_flat)
    return out.reshape(indices.shape + (d,))


# speedup vs baseline: 1.5772x; 1.5772x over previous
import functools

import jax
import jax.numpy as jnp
from jax import lax
from jax.experimental import pallas as pl
from jax.experimental.pallas import tpu as pltpu
from jax.experimental.pallas import tpu_sc as plsc


def _make_gather(b_total: int, d: int):
    info = plsc.get_sparse_core_info()
    nw = info.num_cores * info.num_subcores
    b_per_w = b_total // nw
    chunk = 1024
    n_chunks = b_per_w // chunk

    mesh = plsc.VectorSubcoreMesh(core_axis_name="c", subcore_axis_name="s")

    @functools.partial(
        pl.kernel,
        mesh=mesh,
        out_type=jax.ShapeDtypeStruct((b_total, d), jnp.float32),
        scratch_types=[
            pltpu.VMEM((b_per_w,), jnp.int32),
            pltpu.VMEM((2, chunk, d), jnp.float32),
            pltpu.SemaphoreType.DMA,
            pltpu.SemaphoreType.DMA,
        ],
        compiler_params=pltpu.CompilerParams(use_tc_tiling_on_sc=False),
    )
    def gather_kernel(table_hbm, idx_hbm, out_hbm, idx_v, rows_v, gsem, osem):
        wid = lax.axis_index("s") * info.num_cores + lax.axis_index("c")
        base = wid * b_per_w
        pltpu.sync_copy(idx_hbm.at[pl.ds(base, b_per_w)], idx_v)
        gathers = [None, None]
        outs = [None, None]
        gathers[0] = pltpu.async_copy(
            table_hbm.at[idx_v.at[pl.ds(0, chunk)]], rows_v.at[0], gsem
        )
        for c in range(n_chunks):
            cur = c % 2
            nxt = (c + 1) % 2
            if c + 1 < n_chunks:
                gathers[nxt] = pltpu.async_copy(
                    table_hbm.at[idx_v.at[pl.ds((c + 1) * chunk, chunk)]],
                    rows_v.at[nxt],
                    gsem,
                )
            gathers[cur].wait()
            if outs[cur] is not None:
                outs[cur].wait()
            outs[cur] = pltpu.async_copy(
                rows_v.at[cur],
                out_hbm.at[pl.ds(base + c * chunk, chunk)],
                osem,
            )
        for o in outs:
            if o is not None:
                o.wait()

    return gather_kernel


def kernel(inputs, indices):
    d = inputs.shape[1]
    idx_flat = indices.reshape(-1)
    out = _make_gather(idx_flat.shape[0], d)(inputs, idx_flat)
    return out.reshape(indices.shape + (d,))
